# baseline (speedup 1.0000x reference)
import functools

import jax
import jax.numpy as jnp
from jax import lax
from jax.experimental import pallas as pl
from jax.experimental.pallas import tpu as pltpu
from jax.experimental.pallas import tpu_sc as plsc

_RS = [0.5, 1.0, 1.5, 2.0, 2.5, 3.0, 3.5, 4.0]
_ETA = [4.0, 2.0, 1.0, 0.5, 4.0, 2.0, 1.0, 0.5]
_N_SF = 8

_NC = 2
_NS = 16
_LANES = 16
_NBUF = 5


def _make_sc_fp(n_pairs: int, n_atoms: int, chunk: int):
  n_workers = _NC * _NS
  assert n_pairs % n_workers == 0
  pairs_per_tile = n_pairs // n_workers
  assert pairs_per_tile % chunk == 0
  n_chunks = pairs_per_tile // chunk
  assert n_chunks % _NBUF == 0 and n_chunks > _NBUF
  assert chunk % _LANES == 0 and chunk % 8 == 0

  mesh = plsc.VectorSubcoreMesh(core_axis_name="c", subcore_axis_name="s")

  buf_types = []
  for _ in range(_NBUF):
    buf_types += [
        pltpu.VMEM((chunk,), jnp.float32),
        pltpu.VMEM((chunk,), jnp.float32),
        pltpu.VMEM((chunk, 2), jnp.int32),
        pltpu.VMEM((chunk,), jnp.int32),
        pltpu.VMEM((chunk, 2), jnp.int32),
        pltpu.VMEM((chunk, _N_SF), jnp.float32),
        pltpu.SemaphoreType.DMA,
        pltpu.SemaphoreType.DMA,
        pltpu.SemaphoreType.DMA,
    ]

  @functools.partial(
      pl.kernel,
      out_type=(
          jax.ShapeDtypeStruct((_NC, n_atoms, _N_SF), jnp.float32),
          jax.ShapeDtypeStruct((n_pairs, 2), jnp.int32),
      ),
      mesh=mesh,
      compiler_params=pltpu.CompilerParams(
          needs_layout_passes=False, use_tc_tiling_on_sc=False),
      scratch_types=[pltpu.VMEM_SHARED((n_atoms, _N_SF), jnp.float32)]
      + buf_types,
  )
  def sc_fp(dist_hbm, fc_hbm, ind2_hbm, zero_hbm, part_hbm, jac_hbm,
            acc, *bufs):
    c = lax.axis_index("c")
    s = lax.axis_index("s")
    @pl.when(s == 0)
    def _():
      pltpu.sync_copy(zero_hbm, acc)
    plsc.subcore_barrier()

    wid = s * _NC + c
    base = wid * pairs_per_tile
    iota = lax.iota(jnp.int32, _LANES)
    zero16 = jnp.zeros((_LANES,), dtype=jnp.int32)
    one16 = jnp.ones((_LANES,), dtype=jnp.int32)
    sets = [tuple(bufs[i * 9:(i + 1) * 9]) for i in range(_NBUF)]

    def issue_in(b, k):
      dist_v, fc_v, ind2_v = b[0], b[1], b[2]
      in_sem = b[6]
      off = base + k * chunk
      pltpu.async_copy(dist_hbm.at[pl.ds(off, chunk)], dist_v, in_sem)
      pltpu.async_copy(fc_hbm.at[pl.ds(off, chunk)], fc_v, in_sem)
      pltpu.async_copy(ind2_hbm.at[pl.ds(off, chunk)], ind2_v, in_sem)

    def wait_in(b):
      dist_v, fc_v, ind2_v = b[0], b[1], b[2]
      in_sem = b[6]
      pltpu.make_async_copy(dist_hbm.at[pl.ds(0, chunk)], dist_v, in_sem).wait()
      pltpu.make_async_copy(fc_hbm.at[pl.ds(0, chunk)], fc_v, in_sem).wait()
      pltpu.make_async_copy(ind2_hbm.at[pl.ds(0, chunk)], ind2_v, in_sem).wait()

    def issue_out(b, k):
      idx_v, jac_v, sf_v = b[3], b[4], b[5]
      out_sem, jac_sem = b[7], b[8]
      off = base + k * chunk
      pltpu.async_copy(sf_v, acc.at[idx_v], out_sem, add=True)
      pltpu.async_copy(jac_v, jac_hbm.at[pl.ds(off, chunk)], jac_sem)

    def wait_out(b):
      idx_v, jac_v, sf_v = b[3], b[4], b[5]
      out_sem, jac_sem = b[7], b[8]
      pltpu.make_async_copy(sf_v, acc.at[idx_v], out_sem).wait()
      pltpu.make_async_copy(jac_v, jac_hbm.at[pl.ds(0, chunk)], jac_sem).wait()

    def compute(b, k):
      dist_v, fc_v, ind2_v, idx_v, jac_v, sf_v = b[:6]
      off = base + k * chunk

      @plsc.parallel_loop(0, chunk // _LANES, unroll=4)
      def grp(g):
        p0 = g * _LANES
        rows = p0 + iota
        d = dist_v[pl.ds(p0, _LANES)]
        f = fc_v[pl.ds(p0, _LANES)]
        ii = plsc.load_gather(ind2_v, [rows, zero16])
        idx_v[pl.ds(p0, _LANES)] = ii
        plsc.store_scatter(jac_v, [rows, zero16], off + rows)
        plsc.store_scatter(jac_v, [rows, one16], ii)
        for j in range(_N_SF):
          t = d - _RS[j]
          e = jnp.exp(t * t * (-_ETA[j])) * f
          col = jnp.full((_LANES,), j, dtype=jnp.int32)
          plsc.store_scatter(sf_v, [rows, col], e)

    issue_in(sets[0], 0)

    def outer(kk, carry):
      for p in range(_NBUF):
        k = kk * _NBUF + p
        nxt = sets[(p + 1) % _NBUF]

        @pl.when(k >= _NBUF - 1)
        def _():
          wait_out(nxt)

        @pl.when(k + 1 < n_chunks)
        def _():
          issue_in(nxt, k + 1)

        wait_in(sets[p])
        compute(sets[p], k)
        issue_out(sets[p], k)
      return carry

    lax.fori_loop(0, n_chunks // _NBUF, outer, 0)
    for k in range(n_chunks - (_NBUF - 1), n_chunks):
      wait_out(sets[k % _NBUF])

    plsc.subcore_barrier()
    @pl.when(s == 0)
    def _():
      pltpu.sync_copy(acc, part_hbm.at[c])

  return sc_fp


def _reduce_body(x_ref, o_ref):
  o_ref[...] = x_ref[0] + x_ref[1]


def kernel(dist, fc, ind_2, elems):
  n_pairs = dist.shape[0]
  n_atoms = elems.shape[0]

  chunk = 400
  zeros = jnp.zeros((n_atoms, _N_SF), dtype=jnp.float32)
  sc_fp = _make_sc_fp(n_pairs, n_atoms, chunk)
  partial, jacob_ind = sc_fp(dist, fc, ind_2, zeros)

  flat = partial.reshape(_NC, (n_atoms * _N_SF) // 128, 128)
  fp = pl.pallas_call(
      _reduce_body,
      out_shape=jax.ShapeDtypeStruct(flat.shape[1:], jnp.float32),
  )(flat)
  fp = fp.reshape(n_atoms, _N_SF)

  return fp, jacob_ind


# speedup vs baseline: 1.0524x; 1.0524x over previous
import functools

import jax
import jax.numpy as jnp
from jax import lax
from jax.experimental import pallas as pl
from jax.experimental.pallas import tpu as pltpu
from jax.experimental.pallas import tpu_sc as plsc

_RS = [0.5, 1.0, 1.5, 2.0, 2.5, 3.0, 3.5, 4.0]
_ETA = [4.0, 2.0, 1.0, 0.5, 4.0, 2.0, 1.0, 0.5]
_N_SF = 8

_NC = 2
_NS = 16
_LANES = 16
_NBUF = 5


def _make_sc_fp(n_pairs: int, n_atoms: int, chunk: int):
  n_workers = _NC * _NS
  assert n_pairs % n_workers == 0
  pairs_per_tile = n_pairs // n_workers
  assert pairs_per_tile % chunk == 0
  n_chunks = pairs_per_tile // chunk
  assert n_chunks % _NBUF == 0 and n_chunks > _NBUF
  assert chunk % _LANES == 0 and chunk % 8 == 0

  mesh = plsc.VectorSubcoreMesh(core_axis_name="c", subcore_axis_name="s")

  buf_types = []
  for _ in range(_NBUF):
    buf_types += [
        pltpu.VMEM((chunk,), jnp.float32),
        pltpu.VMEM((chunk,), jnp.float32),
        pltpu.VMEM((2 * chunk,), jnp.int32),
        pltpu.VMEM((chunk,), jnp.int32),
        pltpu.VMEM((2 * chunk,), jnp.int32),
        pltpu.VMEM((chunk, _N_SF), jnp.float32),
        pltpu.SemaphoreType.DMA,
        pltpu.SemaphoreType.DMA,
        pltpu.SemaphoreType.DMA,
    ]

  @functools.partial(
      pl.kernel,
      out_type=(
          jax.ShapeDtypeStruct((_NC, n_atoms, _N_SF), jnp.float32),
          jax.ShapeDtypeStruct((2 * n_pairs,), jnp.int32),
      ),
      mesh=mesh,
      compiler_params=pltpu.CompilerParams(
          needs_layout_passes=False, use_tc_tiling_on_sc=False),
      scratch_types=[pltpu.VMEM_SHARED((n_atoms, _N_SF), jnp.float32)]
      + buf_types,
  )
  def sc_fp(dist_hbm, fc_hbm, ind2_hbm, zero_hbm, part_hbm, jac_hbm,
            acc, *bufs):
    c = lax.axis_index("c")
    s = lax.axis_index("s")
    @pl.when(s == 0)
    def _():
      pltpu.sync_copy(zero_hbm, acc)
    plsc.subcore_barrier()

    wid = s * _NC + c
    base = wid * pairs_per_tile
    iota = lax.iota(jnp.int32, _LANES)
    zero16 = jnp.zeros((_LANES,), dtype=jnp.int32)
    one16 = jnp.ones((_LANES,), dtype=jnp.int32)
    sets = [tuple(bufs[i * 9:(i + 1) * 9]) for i in range(_NBUF)]

    def issue_in(b, k):
      dist_v, fc_v, ind2_v = b[0], b[1], b[2]
      in_sem = b[6]
      off = base + k * chunk
      pltpu.async_copy(dist_hbm.at[pl.ds(off, chunk)], dist_v, in_sem)
      pltpu.async_copy(fc_hbm.at[pl.ds(off, chunk)], fc_v, in_sem)
      pltpu.async_copy(ind2_hbm.at[pl.ds(2 * off, 2 * chunk)], ind2_v, in_sem)

    def wait_in(b):
      dist_v, fc_v, ind2_v = b[0], b[1], b[2]
      in_sem = b[6]
      pltpu.make_async_copy(dist_hbm.at[pl.ds(0, chunk)], dist_v, in_sem).wait()
      pltpu.make_async_copy(fc_hbm.at[pl.ds(0, chunk)], fc_v, in_sem).wait()
      pltpu.make_async_copy(ind2_hbm.at[pl.ds(0, 2 * chunk)], ind2_v,
                            in_sem).wait()

    def issue_out(b, k):
      idx_v, jac_v, sf_v = b[3], b[4], b[5]
      out_sem, jac_sem = b[7], b[8]
      off = base + k * chunk
      pltpu.async_copy(sf_v, acc.at[idx_v], out_sem, add=True)
      pltpu.async_copy(jac_v, jac_hbm.at[pl.ds(2 * off, 2 * chunk)], jac_sem)

    def wait_out(b):
      idx_v, jac_v, sf_v = b[3], b[4], b[5]
      out_sem, jac_sem = b[7], b[8]
      pltpu.make_async_copy(sf_v, acc.at[idx_v], out_sem).wait()
      pltpu.make_async_copy(jac_v, jac_hbm.at[pl.ds(0, 2 * chunk)],
                            jac_sem).wait()

    def compute(b, k):
      dist_v, fc_v, ind2_v, idx_v, jac_v, sf_v = b[:6]
      off = base + k * chunk

      @plsc.parallel_loop(0, chunk // _LANES, unroll=4)
      def grp(g):
        p0 = g * _LANES
        rows = p0 + iota
        d = dist_v[pl.ds(p0, _LANES)]
        f = fc_v[pl.ds(p0, _LANES)]
        rows2 = rows + rows
        ii = plsc.load_gather(ind2_v, [rows2])
        idx_v[pl.ds(p0, _LANES)] = ii
        plsc.store_scatter(jac_v, [rows2], off + rows)
        plsc.store_scatter(jac_v, [rows2 + one16], ii)
        for j in range(_N_SF):
          t = d - _RS[j]
          e = jnp.exp(t * t * (-_ETA[j])) * f
          col = jnp.full((_LANES,), j, dtype=jnp.int32)
          plsc.store_scatter(sf_v, [rows, col], e)

    issue_in(sets[0], 0)

    def outer(kk, carry):
      for p in range(_NBUF):
        k = kk * _NBUF + p
        nxt = sets[(p + 1) % _NBUF]

        @pl.when(k >= _NBUF - 1)
        def _():
          wait_out(nxt)

        @pl.when(k + 1 < n_chunks)
        def _():
          issue_in(nxt, k + 1)

        wait_in(sets[p])
        compute(sets[p], k)
        issue_out(sets[p], k)
      return carry

    lax.fori_loop(0, n_chunks // _NBUF, outer, 0)
    for k in range(n_chunks - (_NBUF - 1), n_chunks):
      wait_out(sets[k % _NBUF])

    plsc.subcore_barrier()
    @pl.when(s == 0)
    def _():
      pltpu.sync_copy(acc, part_hbm.at[c])

  return sc_fp


def _reduce_body(x_ref, o_ref):
  o_ref[...] = x_ref[0] + x_ref[1]


def kernel(dist, fc, ind_2, elems):
  n_pairs = dist.shape[0]
  n_atoms = elems.shape[0]

  chunk = 400
  zeros = jnp.zeros((n_atoms, _N_SF), dtype=jnp.float32)
  sc_fp = _make_sc_fp(n_pairs, n_atoms, chunk)
  partial, jac_flat = sc_fp(dist, fc, ind_2.reshape(-1), zeros)
  jacob_ind = jac_flat.reshape(n_pairs, 2)

  flat = partial.reshape(_NC, (n_atoms * _N_SF) // 128, 128)
  fp = pl.pallas_call(
      _reduce_body,
      out_shape=jax.ShapeDtypeStruct(flat.shape[1:], jnp.float32),
  )(flat)
  fp = fp.reshape(n_atoms, _N_SF)

  return fp, jacob_ind


# speedup vs baseline: 26.8809x; 25.5435x over previous
import functools

import jax
import jax.numpy as jnp
from jax import lax
from jax.experimental import pallas as pl
from jax.experimental.pallas import tpu as pltpu
from jax.experimental.pallas import tpu_sc as plsc

_RS = [0.5, 1.0, 1.5, 2.0, 2.5, 3.0, 3.5, 4.0]
_ETA = [4.0, 2.0, 1.0, 0.5, 4.0, 2.0, 1.0, 0.5]
_N_SF = 8

_NC = 2
_NS = 16
_LANES = 16
_NBUF = 5


def _make_sc_fp(n_pairs: int, n_atoms: int, chunk: int):
  n_workers = _NC * _NS
  assert n_pairs % n_workers == 0
  pairs_per_tile = n_pairs // n_workers
  assert pairs_per_tile % chunk == 0
  n_chunks = pairs_per_tile // chunk
  assert n_chunks % _NBUF == 0 and n_chunks > _NBUF
  assert chunk % _LANES == 0 and chunk % 8 == 0

  mesh = plsc.VectorSubcoreMesh(core_axis_name="c", subcore_axis_name="s")

  buf_types = []
  for _ in range(_NBUF):
    buf_types += [
        pltpu.VMEM((chunk,), jnp.float32),
        pltpu.VMEM((chunk,), jnp.float32),
        pltpu.VMEM((chunk,), jnp.int32),
        pltpu.VMEM((chunk, _N_SF), jnp.float32),
        pltpu.SemaphoreType.DMA,
        pltpu.SemaphoreType.DMA,
    ]

  @functools.partial(
      pl.kernel,
      out_type=jax.ShapeDtypeStruct((_NC, n_atoms, _N_SF), jnp.float32),
      mesh=mesh,
      compiler_params=pltpu.CompilerParams(
          needs_layout_passes=False, use_tc_tiling_on_sc=False),
      scratch_types=[pltpu.VMEM_SHARED((n_atoms, _N_SF), jnp.float32)]
      + buf_types,
  )
  def sc_fp(dist_hbm, fc_hbm, idx_hbm, zero_hbm, part_hbm, acc, *bufs):
    c = lax.axis_index("c")
    s = lax.axis_index("s")
    @pl.when(s == 0)
    def _():
      pltpu.sync_copy(zero_hbm, acc)
    plsc.subcore_barrier()

    wid = s * _NC + c
    base = wid * pairs_per_tile
    iota = lax.iota(jnp.int32, _LANES)
    sets = [tuple(bufs[i * 6:(i + 1) * 6]) for i in range(_NBUF)]

    def issue_in(b, k):
      dist_v, fc_v, idx_v, _, in_sem, _ = b
      off = base + k * chunk
      pltpu.async_copy(dist_hbm.at[pl.ds(off, chunk)], dist_v, in_sem)
      pltpu.async_copy(fc_hbm.at[pl.ds(off, chunk)], fc_v, in_sem)
      pltpu.async_copy(idx_hbm.at[pl.ds(off, chunk)], idx_v, in_sem)

    def wait_in(b):
      dist_v, fc_v, idx_v, _, in_sem, _ = b
      pltpu.make_async_copy(dist_hbm.at[pl.ds(0, chunk)], dist_v, in_sem).wait()
      pltpu.make_async_copy(fc_hbm.at[pl.ds(0, chunk)], fc_v, in_sem).wait()
      pltpu.make_async_copy(idx_hbm.at[pl.ds(0, chunk)], idx_v, in_sem).wait()

    def issue_stream(b):
      _, _, idx_v, sf_v, _, out_sem = b
      pltpu.async_copy(sf_v, acc.at[idx_v], out_sem, add=True)

    def wait_stream(b):
      _, _, idx_v, sf_v, _, out_sem = b
      pltpu.make_async_copy(sf_v, acc.at[idx_v], out_sem).wait()

    def compute(b):
      dist_v, fc_v, _, sf_v, _, _ = b

      @plsc.parallel_loop(0, chunk // _LANES, unroll=4)
      def grp(g):
        p0 = g * _LANES
        d = dist_v[pl.ds(p0, _LANES)]
        f = fc_v[pl.ds(p0, _LANES)]
        rows = p0 + iota
        for j in range(_N_SF):
          t = d - _RS[j]
          e = jnp.exp(t * t * (-_ETA[j])) * f
          col = jnp.full((_LANES,), j, dtype=jnp.int32)
          plsc.store_scatter(sf_v, [rows, col], e)

    issue_in(sets[0], 0)

    def outer(kk, carry):
      for p in range(_NBUF):
        k = kk * _NBUF + p
        nxt = sets[(p + 1) % _NBUF]

        @pl.when(k >= _NBUF - 1)
        def _():
          wait_stream(nxt)

        @pl.when(k + 1 < n_chunks)
        def _():
          issue_in(nxt, k + 1)

        wait_in(sets[p])
        compute(sets[p])
        issue_stream(sets[p])
      return carry

    lax.fori_loop(0, n_chunks // _NBUF, outer, 0)
    for k in range(n_chunks - (_NBUF - 1), n_chunks):
      wait_stream(sets[k % _NBUF])

    plsc.subcore_barrier()
    @pl.when(s == 0)
    def _():
      pltpu.sync_copy(acc, part_hbm.at[c])

  return sc_fp


def _reduce_body(x_ref, o_ref):
  o_ref[...] = x_ref[0] + x_ref[1]


def kernel(dist, fc, ind_2, elems):
  n_pairs = dist.shape[0]
  n_atoms = elems.shape[0]
  i_rind = ind_2[:, 0]

  chunk = 800
  zeros = jnp.zeros((n_atoms, _N_SF), dtype=jnp.float32)
  sc_fp = _make_sc_fp(n_pairs, n_atoms, chunk)
  partial = sc_fp(dist, fc, i_rind, zeros)

  flat = partial.reshape(_NC, (n_atoms * _N_SF) // 128, 128)
  fp = pl.pallas_call(
      _reduce_body,
      out_shape=jax.ShapeDtypeStruct(flat.shape[1:], jnp.float32),
  )(flat)
  fp = fp.reshape(n_atoms, _N_SF)

  p_ind = jnp.arange(n_pairs, dtype=jnp.int32)
  jacob_ind = jnp.stack([p_ind, i_rind], axis=1)
  return fp, jacob_ind
